# Initial kernel scaffold; baseline (speedup 1.0000x reference)
import jax
import jax.numpy as jnp
from jax.experimental import pallas as pl


def kernel(feats, W_conv, b_conv, C):
    raise NotImplementedError("write your pallas kernel here")


# speedup vs baseline: 47.1554x; 47.1554x over previous
import functools

import jax
import jax.numpy as jnp
from jax.experimental import pallas as pl
from jax.experimental.pallas import tpu as pltpu

B = 4
C_IN = 448
H = 96
W = 96
D_OUT = 28
N_CENTERS = 2304
HW = H * W

NC = 4
CCHUNK = C_IN // NC
ROWS = 768
NT = HW // ROWS
RH = ROWS // W

_BIG_F = 3e38
_BIG_I = 1 << 30


def _dsvdd_kernel(feats_ref, wt_ref, wconv_ref, bias_ref, c_ref,
                  out_ref, phi_acc):
    c = pl.program_id(1)

    f = feats_ref[0, 0].reshape(CCHUNK, HW)
    wt = wt_ref[0]
    part = jax.lax.dot_general(
        wt, f, (((0,), (0,)), ((), ())),
        preferred_element_type=jnp.float32)

    @pl.when(c == 0)
    def _init():
        phi_acc[...] = part

    @pl.when(c > 0)
    def _acc():
        phi_acc[...] = phi_acc[...] + part

    @pl.when(c == NC - 1)
    def _finish():
        x = phi_acc[...].reshape(D_OUT, H, W)
        zw = jnp.zeros((D_OUT, H, 1), jnp.float32)
        xw = (x
              + jnp.concatenate([zw, x[:, :, :W - 1]], axis=2)
              + jnp.concatenate([x[:, :, 1:], zw], axis=2))
        zh = jnp.zeros((D_OUT, 1, W), jnp.float32)
        xs = (xw
              + jnp.concatenate([zh, xw[:, :H - 1, :]], axis=1)
              + jnp.concatenate([xw[:, 1:, :], zh], axis=1))
        pooled = xs * jnp.float32(1.0 / 9.0)

        wx = wconv_ref[:, C_IN:C_IN + 1]
        wy = wconv_ref[:, C_IN + 1:C_IN + 2]
        xx = (jax.lax.broadcasted_iota(jnp.int32, (1, H, W), 1)
              .astype(jnp.float32) / jnp.float32(H - 1)) * 2.0 - 1.0
        yy = (jax.lax.broadcasted_iota(jnp.int32, (1, H, W), 2)
              .astype(jnp.float32) / jnp.float32(W - 1)) * 2.0 - 1.0
        phi = (pooled + wx[:, :, None] * xx + wy[:, :, None] * yy
               + bias_ref[...][:, :, None])
        phi_flat = phi.reshape(D_OUT, HW)

        cmat = c_ref[...]
        c2 = jnp.sum(cmat * cmat, axis=0, keepdims=True)

        for t in range(NT):
            sl = phi_flat[:, t * ROWS:(t + 1) * ROWS]
            x2 = jnp.sum(sl * sl, axis=0).reshape(ROWS, 1)
            g = jax.lax.dot_general(
                sl, cmat, (((0,), (0,)), ((), ())),
                preferred_element_type=jnp.float32)
            d = c2 - 2.0 * g
            iota = jax.lax.broadcasted_iota(jnp.int32, (ROWS, N_CENTERS), 1)

            m1 = jnp.min(d, axis=1, keepdims=True)
            a1 = jnp.min(jnp.where(d == m1, iota, _BIG_I), axis=1,
                         keepdims=True)
            d = jnp.where(iota == a1, _BIG_F, d)
            m2 = jnp.min(d, axis=1, keepdims=True)
            a2 = jnp.min(jnp.where(d == m2, iota, _BIG_I), axis=1,
                         keepdims=True)
            d = jnp.where(iota == a2, _BIG_F, d)
            m3 = jnp.min(d, axis=1, keepdims=True)

            eps = jnp.float32(1e-12)
            d0 = jnp.sqrt(jnp.maximum(m1 + x2, eps))
            d1 = jnp.sqrt(jnp.maximum(m2 + x2, eps))
            d2 = jnp.sqrt(jnp.maximum(m3 + x2, eps))
            e0 = jnp.exp(-d0)
            e1 = jnp.exp(-d1)
            e2 = jnp.exp(-d2)
            score = d0 * e0 / (e0 + e1 + e2)
            out_ref[0, 0, t * RH:(t + 1) * RH, :] = score.reshape(RH, W)


@jax.jit
def kernel(feats, W_conv, b_conv, C):
    wt = W_conv[:, :C_IN].T.reshape(NC, CCHUNK, D_OUT)
    bias = b_conv.reshape(D_OUT, 1)
    grid = (B, NC)
    return pl.pallas_call(
        _dsvdd_kernel,
        grid=grid,
        in_specs=[
            pl.BlockSpec((1, 1, CCHUNK, H, W), lambda b, c: (0, b, c, 0, 0)),
            pl.BlockSpec((1, CCHUNK, D_OUT), lambda b, c: (c, 0, 0)),
            pl.BlockSpec((D_OUT, C_IN + 2), lambda b, c: (0, 0)),
            pl.BlockSpec((D_OUT, 1), lambda b, c: (0, 0)),
            pl.BlockSpec((D_OUT, N_CENTERS), lambda b, c: (0, 0)),
        ],
        out_specs=pl.BlockSpec((1, 1, H, W), lambda b, c: (b, 0, 0, 0)),
        out_shape=jax.ShapeDtypeStruct((B, 1, H, W), jnp.float32),
        scratch_shapes=[pltpu.VMEM((D_OUT, HW), jnp.float32)],
        compiler_params=pltpu.CompilerParams(
            dimension_semantics=("arbitrary", "arbitrary"),
        ),
    )(feats, wt, W_conv, bias, C)


# speedup vs baseline: 48.0040x; 1.0180x over previous
import functools

import jax
import jax.numpy as jnp
from jax.experimental import pallas as pl
from jax.experimental.pallas import tpu as pltpu

B = 4
C_IN = 448
H = 96
W = 96
D_OUT = 28
N_CENTERS = 2304
HW = H * W

DA = 32
NC = 4
CCHUNK = C_IN // NC
ROWS = 768
NT = HW // ROWS
RH = ROWS // W

_BIG_F = 3e38
_BIG_I = 1 << 30


def _dsvdd_kernel(feats_ref, wt_ref, wconv_ref, bias_ref, c_ref,
                  out_ref, phi_acc):
    c = pl.program_id(1)

    f = feats_ref[0, 0].reshape(CCHUNK, HW)
    wt = wt_ref[0]
    part = jax.lax.dot_general(
        wt, f, (((0,), (0,)), ((), ())),
        preferred_element_type=jnp.float32)

    @pl.when(c == 0)
    def _init():
        phi_acc[...] = part

    @pl.when(c > 0)
    def _acc():
        phi_acc[...] = phi_acc[...] + part

    @pl.when(c == NC - 1)
    def _finish():
        x = phi_acc[...].reshape(D_OUT, H, W)
        zw = jnp.zeros((D_OUT, H, 1), jnp.float32)
        xw = (x
              + jnp.concatenate([zw, x[:, :, :W - 1]], axis=2)
              + jnp.concatenate([x[:, :, 1:], zw], axis=2))
        zh = jnp.zeros((D_OUT, 1, W), jnp.float32)
        xs = (xw
              + jnp.concatenate([zh, xw[:, :H - 1, :]], axis=1)
              + jnp.concatenate([xw[:, 1:, :], zh], axis=1))
        pooled = xs * jnp.float32(1.0 / 9.0)

        wx = wconv_ref[:, C_IN:C_IN + 1]
        wy = wconv_ref[:, C_IN + 1:C_IN + 2]
        xx = (jax.lax.broadcasted_iota(jnp.int32, (1, H, W), 1)
              .astype(jnp.float32) / jnp.float32(H - 1)) * 2.0 - 1.0
        yy = (jax.lax.broadcasted_iota(jnp.int32, (1, H, W), 2)
              .astype(jnp.float32) / jnp.float32(W - 1)) * 2.0 - 1.0
        phi = (pooled + wx[:, :, None] * xx + wy[:, :, None] * yy
               + bias_ref[...][:, :, None])
        phi_flat = phi.reshape(D_OUT, HW)
        phi_aug = jnp.concatenate(
            [phi_flat, jnp.ones((1, HW), jnp.float32),
             jnp.zeros((DA - D_OUT - 1, HW), jnp.float32)], axis=0)

        caug = c_ref[...]

        for t in range(NT):
            sl = phi_aug[:, t * ROWS:(t + 1) * ROWS]
            slp = sl[:D_OUT]
            x2 = jnp.sum(slp * slp, axis=0).reshape(ROWS, 1)
            d = jax.lax.dot_general(
                sl, caug, (((0,), (0,)), ((), ())),
                preferred_element_type=jnp.float32)

            m1 = jnp.min(d, axis=1, keepdims=True)
            gt1 = d > m1
            n_gt1 = jnp.sum(gt1.astype(jnp.float32), axis=1, keepdims=True)
            m2 = jnp.min(jnp.where(gt1, d, _BIG_F), axis=1, keepdims=True)
            gt2 = d > m2
            n_gt2 = jnp.sum(gt2.astype(jnp.float32), axis=1, keepdims=True)
            m3 = jnp.min(jnp.where(gt2, d, _BIG_F), axis=1, keepdims=True)

            c1 = jnp.float32(N_CENTERS) - n_gt1
            c2n = n_gt1 - n_gt2
            second = jnp.where(c1 >= 2.0, m1, m2)
            third = jnp.where(
                c1 >= 3.0, m1,
                jnp.where(c1 >= 2.0, m2,
                          jnp.where(c2n >= 2.0, m2, m3)))

            eps = jnp.float32(1e-12)
            d0 = jnp.sqrt(jnp.maximum(m1 + x2, eps))
            d1 = jnp.sqrt(jnp.maximum(second + x2, eps))
            d2 = jnp.sqrt(jnp.maximum(third + x2, eps))
            e0 = jnp.exp(-d0)
            e1 = jnp.exp(-d1)
            e2 = jnp.exp(-d2)
            score = d0 * e0 / (e0 + e1 + e2)
            out_ref[0, 0, t * RH:(t + 1) * RH, :] = score.reshape(RH, W)


@jax.jit
def kernel(feats, W_conv, b_conv, C):
    wt = W_conv[:, :C_IN].T.reshape(NC, CCHUNK, D_OUT)
    bias = b_conv.reshape(D_OUT, 1)
    caug = jnp.concatenate(
        [-2.0 * C, jnp.sum(C * C, axis=0, keepdims=True),
         jnp.zeros((DA - D_OUT - 1, N_CENTERS), jnp.float32)], axis=0)
    grid = (B, NC)
    return pl.pallas_call(
        _dsvdd_kernel,
        grid=grid,
        in_specs=[
            pl.BlockSpec((1, 1, CCHUNK, H, W), lambda b, c: (0, b, c, 0, 0)),
            pl.BlockSpec((1, CCHUNK, D_OUT), lambda b, c: (c, 0, 0)),
            pl.BlockSpec((D_OUT, C_IN + 2), lambda b, c: (0, 0)),
            pl.BlockSpec((D_OUT, 1), lambda b, c: (0, 0)),
            pl.BlockSpec((DA, N_CENTERS), lambda b, c: (0, 0)),
        ],
        out_specs=pl.BlockSpec((1, 1, H, W), lambda b, c: (b, 0, 0, 0)),
        out_shape=jax.ShapeDtypeStruct((B, 1, H, W), jnp.float32),
        scratch_shapes=[pltpu.VMEM((D_OUT, HW), jnp.float32)],
        compiler_params=pltpu.CompilerParams(
            dimension_semantics=("arbitrary", "arbitrary"),
        ),
    )(feats, wt, W_conv, bias, caug)
